# Initial kernel scaffold; baseline (speedup 1.0000x reference)
import jax
import jax.numpy as jnp
from jax.experimental import pallas as pl


def kernel(data, edge_index_list, W1, a1s, a1d, b1, W2, a2s, a2d, b2, Watt, batt, uatt, Wf, bf):
    raise NotImplementedError("write your pallas kernel here")


# speedup vs baseline: 18.2086x; 18.2086x over previous
import functools

import jax
import jax.numpy as jnp
from jax import lax
from jax.experimental import pallas as pl
from jax.experimental.pallas import tpu as pltpu
from jax.experimental.pallas import tpu_sc as plsc

N = 10000
E = 160000
D_IN = 256
HID = 64
HEADS = 4
P = 2
BOTTLE = 64
ATT = 64

F32 = jnp.float32

NUM_CORES = 2
NUM_SUB = 16
CHUNK = 80
EDGES_PER_SUB = E // NUM_SUB
NCHUNK = EDGES_PER_SUB // CHUNK
ROWS_PER_SUB = N // NUM_SUB
NPAD = 10112
BCHUNK = 40
BEDGES_PER_SUB = E // 2 // NUM_SUB
NBCHUNK = BEDGES_PER_SUB // BCHUNK

BN = 1000
GRID_N = N // BN

_mesh = plsc.VectorSubcoreMesh(core_axis_name="c", subcore_axis_name="s")


def _sc_agg(ha, hb, es16, ed16, src, dst):
    @functools.partial(
        pl.kernel,
        out_type=[
            jax.ShapeDtypeStruct((N, 128), F32),
            jax.ShapeDtypeStruct((N, 128), F32),
            jax.ShapeDtypeStruct((N, 128), F32),
            jax.ShapeDtypeStruct((N, 128), F32),
        ],
        mesh=_mesh,
        scratch_types=[
            pltpu.VMEM((1, CHUNK), jnp.int32),
            pltpu.VMEM((1, CHUNK), jnp.int32),
            pltpu.VMEM((CHUNK, 128), F32),
            pltpu.VMEM((CHUNK, 128), F32),
            pltpu.VMEM((CHUNK, 128), F32),
            pltpu.VMEM((CHUNK, 128), F32),
            pltpu.VMEM_SHARED((NPAD, 128), F32),
        ],
    )
    def k(ha_h, hb_h, es_h, ed_h, src_h, dst_h, sa_o, sb_o, d0_o, d1_o,
          sidx, didx, esr, edr, hrow, srow, acc):
        c = lax.axis_index("c")
        s = lax.axis_index("s")
        zero16 = jnp.zeros((16,), F32)

        rst = s * ROWS_PER_SUB // 8 * 8

        def zero_acc():
            @pl.loop(0, CHUNK)
            def _(i):
                for r in range(8):
                    hrow[i, pl.ds(r * 16, 16)] = zero16

            for k2 in range(8):
                pltpu.sync_copy(hrow, acc.at[pl.ds(rst + k2 * CHUNK, CHUNK)])

        zero_acc()
        plsc.subcore_barrier()

        idx0 = jnp.full((16,), 2 * c, jnp.int32)
        idx1 = idx0 + 1

        @pl.loop(0, NCHUNK)
        def _(kk):
            off = s * EDGES_PER_SUB + kk * CHUNK
            pltpu.sync_copy(src_h.at[pl.ds(off, CHUNK)], sidx.at[0])
            pltpu.sync_copy(dst_h.at[pl.ds(off, CHUNK)], didx.at[0])
            pltpu.sync_copy(es_h.at[sidx.at[0]], esr)
            pltpu.sync_copy(ed_h.at[didx.at[0]], edr)

            @pl.when(c == 0)
            def _():
                pltpu.sync_copy(ha_h.at[sidx.at[0]], hrow)

            @pl.when(c == 1)
            def _():
                pltpu.sync_copy(hb_h.at[sidx.at[0]], hrow)

            @pl.loop(0, CHUNK)
            def _(i):
                e = esr[i, pl.ds(0, 16)] + edr[i, pl.ds(0, 16)]
                e = jnp.maximum(e, 0.2 * e)
                ex = jnp.exp(e)
                a0 = ex.at[idx0].get(mode="promise_in_bounds")
                a1 = ex.at[idx1].get(mode="promise_in_bounds")
                for r in range(8):
                    av = a0 if r < 4 else a1
                    srow[i, pl.ds(r * 16, 16)] = hrow[i, pl.ds(r * 16, 16)] * av

            pltpu.sync_copy(srow, acc.at[didx.at[0]], add=True)

        plsc.subcore_barrier()

        @pl.when(c == 0)
        def _():
            pltpu.sync_copy(acc.at[pl.ds(rst, 632)], sa_o.at[pl.ds(rst, 632)])

        @pl.when(c == 1)
        def _():
            pltpu.sync_copy(acc.at[pl.ds(rst, 632)], sb_o.at[pl.ds(rst, 632)])

        plsc.subcore_barrier()
        zero_acc()
        plsc.subcore_barrier()

        @pl.loop(0, NBCHUNK)
        def _(kk):
            off = c * (E // 2) + s * BEDGES_PER_SUB + kk * BCHUNK
            pltpu.sync_copy(src_h.at[pl.ds(off, BCHUNK)],
                            sidx.at[0, pl.ds(0, BCHUNK)])
            pltpu.sync_copy(dst_h.at[pl.ds(off, BCHUNK)],
                            didx.at[0, pl.ds(0, BCHUNK)])
            pltpu.sync_copy(es_h.at[sidx.at[0, pl.ds(0, BCHUNK)]],
                            esr.at[pl.ds(0, BCHUNK)])
            pltpu.sync_copy(ed_h.at[didx.at[0, pl.ds(0, BCHUNK)]],
                            edr.at[pl.ds(0, BCHUNK)])

            @pl.loop(0, BCHUNK)
            def _(i):
                e = esr[i, pl.ds(0, 16)] + edr[i, pl.ds(0, 16)]
                e = jnp.maximum(e, 0.2 * e)
                srow[i, pl.ds(0, 16)] = jnp.exp(e)

            pltpu.sync_copy(srow.at[pl.ds(0, BCHUNK)],
                            acc.at[didx.at[0, pl.ds(0, BCHUNK)]], add=True)

        plsc.subcore_barrier()

        @pl.when(c == 0)
        def _():
            pltpu.sync_copy(acc.at[pl.ds(rst, 632)], d0_o.at[pl.ds(rst, 632)])

        @pl.when(c == 1)
        def _():
            pltpu.sync_copy(acc.at[pl.ds(rst, 632)], d1_o.at[pl.ds(rst, 632)])

    return k(ha, hb, es16, ed16, src, dst)


def _tc_dense(x, wcat, as16, ad16):
    def body(x_ref, w_ref, as_ref, ad_ref, ha_ref, hb_ref, es_ref, ed_ref):
        h = jnp.dot(x_ref[...], w_ref[...], preferred_element_type=F32)
        ha_ref[...] = h[:, :128]
        hb_ref[...] = h[:, 128:]
        es_ref[...] = jnp.dot(h, as_ref[...], preferred_element_type=F32)
        ed_ref[...] = jnp.dot(h, ad_ref[...], preferred_element_type=F32)

    return pl.pallas_call(
        body,
        grid=(GRID_N,),
        in_specs=[
            pl.BlockSpec((BN, D_IN), lambda i: (i, 0)),
            pl.BlockSpec((D_IN, 256), lambda i: (0, 0)),
            pl.BlockSpec((256, 128), lambda i: (0, 0)),
            pl.BlockSpec((256, 128), lambda i: (0, 0)),
        ],
        out_specs=[
            pl.BlockSpec((BN, 128), lambda i: (i, 0)),
            pl.BlockSpec((BN, 128), lambda i: (i, 0)),
            pl.BlockSpec((BN, 128), lambda i: (i, 0)),
            pl.BlockSpec((BN, 128), lambda i: (i, 0)),
        ],
        out_shape=[
            jax.ShapeDtypeStruct((N, 128), F32),
            jax.ShapeDtypeStruct((N, 128), F32),
            jax.ShapeDtypeStruct((N, 128), F32),
            jax.ShapeDtypeStruct((N, 128), F32),
        ],
    )(x, wcat, as16, ad16)


def _tc_finalize(sa, sb, d0, d1, b256):
    def body(sa_ref, sb_ref, d0_ref, d1_ref, b_ref, o_ref):
        d = d0_ref[...] + d1_ref[...]
        segs = []
        for j in range(4):
            srcr = sa_ref if j < 2 else sb_ref
            jj = j % 2
            seg = srcr[...][:, jj * 64:(jj + 1) * 64]
            segs.append(seg / (d[:, j:j + 1] + 1e-16))
        x = jnp.concatenate(segs, axis=1) + b_ref[...]
        o_ref[...] = jnp.where(x > 0, x, jnp.exp(jnp.minimum(x, 0.0)) - 1.0)

    return pl.pallas_call(
        body,
        grid=(GRID_N,),
        in_specs=[
            pl.BlockSpec((BN, 128), lambda i: (i, 0)),
            pl.BlockSpec((BN, 128), lambda i: (i, 0)),
            pl.BlockSpec((BN, 128), lambda i: (i, 0)),
            pl.BlockSpec((BN, 128), lambda i: (i, 0)),
            pl.BlockSpec((1, 256), lambda i: (0, 0)),
        ],
        out_specs=pl.BlockSpec((BN, 256), lambda i: (i, 0)),
        out_shape=jax.ShapeDtypeStruct((N, 256), F32),
    )(sa, sb, d0, d1, b256)


def _tc_pool(h0, h1, watt, batt2, uatt2, wf, bf):
    def body(h0_ref, h1_ref, wa_ref, ba_ref, ua_ref, wf_ref, bf_ref,
             bot_ref, fin_ref):
        h0b = h0_ref[...]
        h1b = h1_ref[...]
        wa = wa_ref[...]
        ba = ba_ref[...]
        ua = ua_ref[...]
        v0 = jnp.tanh(jnp.dot(h0b, wa, preferred_element_type=F32) + ba)
        v1 = jnp.tanh(jnp.dot(h1b, wa, preferred_element_type=F32) + ba)
        vu0 = jnp.dot(v0, ua, preferred_element_type=F32)
        vu1 = jnp.dot(v1, ua, preferred_element_type=F32)
        m = jnp.maximum(vu0, vu1)
        e0 = jnp.exp(vu0 - m)
        e1 = jnp.exp(vu1 - m)
        tot = e0 + e1
        fin = (e0 / tot) * h0b + (e1 / tot) * h1b
        fin_ref[...] = fin
        wfm = jnp.mean(wf_ref[...], axis=0)
        bfm = jnp.mean(bf_ref[...], axis=0, keepdims=True)
        bot_ref[...] = jnp.dot(fin, wfm, preferred_element_type=F32) + bfm

    return pl.pallas_call(
        body,
        grid=(GRID_N,),
        in_specs=[
            pl.BlockSpec((BN, 256), lambda i: (i, 0)),
            pl.BlockSpec((BN, 256), lambda i: (i, 0)),
            pl.BlockSpec((256, ATT), lambda i: (0, 0)),
            pl.BlockSpec((1, ATT), lambda i: (0, 0)),
            pl.BlockSpec((ATT, 1), lambda i: (0, 0)),
            pl.BlockSpec((HEADS, 256, BOTTLE), lambda i: (0, 0, 0)),
            pl.BlockSpec((HEADS, BOTTLE), lambda i: (0, 0)),
        ],
        out_specs=[
            pl.BlockSpec((BN, BOTTLE), lambda i: (i, 0)),
            pl.BlockSpec((BN, 256), lambda i: (i, 0)),
        ],
        out_shape=[
            jax.ShapeDtypeStruct((N, BOTTLE), F32),
            jax.ShapeDtypeStruct((N, 256), F32),
        ],
    )(h0, h1, watt, batt2, uatt2, wf, bf)


def _pack_att(a):
    oh = jax.nn.one_hot(jnp.arange(HEADS), 128, dtype=F32)
    return (a[:, :, None] * oh[:, None, :]).reshape(HEADS * HID, 128)


def kernel(data, edge_index_list, W1, a1s, a1d, b1, W2, a2s, a2d, b2,
           Watt, batt, uatt, Wf, bf):
    w1cat = jnp.transpose(W1, (1, 0, 2)).reshape(D_IN, HEADS * HID)
    w2cat = jnp.transpose(W2, (1, 0, 2)).reshape(HEADS * HID, HEADS * HID)
    as1, ad1 = _pack_att(a1s), _pack_att(a1d)
    as2, ad2 = _pack_att(a2s), _pack_att(a2d)
    b1r = b1.reshape(1, HEADS * HID)
    b2r = b2.reshape(1, HEADS * HID)

    ha1, hb1, es1, ed1 = _tc_dense(data, w1cat, as1, ad1)

    h2 = []
    for p in range(P):
        src = edge_index_list[p, 0]
        dst = edge_index_list[p, 1]
        sa, sb, d0, d1 = _sc_agg(ha1, hb1, es1, ed1, src, dst)
        h1p = _tc_finalize(sa, sb, d0, d1, b1r)
        ha2, hb2, es2, ed2 = _tc_dense(h1p, w2cat, as2, ad2)
        sa2, sb2, d02, d12 = _sc_agg(ha2, hb2, es2, ed2, src, dst)
        h2.append(_tc_finalize(sa2, sb2, d02, d12, b2r))

    bottle, final = _tc_pool(h2[0], h2[1], Watt, batt.reshape(1, ATT),
                             uatt.reshape(ATT, 1), Wf, bf)
    return (bottle, final)


# speedup vs baseline: 20.0506x; 1.1012x over previous
import functools

import jax
import jax.numpy as jnp
from jax import lax
from jax.experimental import pallas as pl
from jax.experimental.pallas import tpu as pltpu
from jax.experimental.pallas import tpu_sc as plsc

N = 10000
E = 160000
D_IN = 256
HID = 64
HEADS = 4
P = 2
BOTTLE = 64
ATT = 64

F32 = jnp.float32

NUM_CORES = 2
NUM_SUB = 16
CHUNK = 80
EDGES_PER_SUB = E // NUM_SUB
NCHUNK = EDGES_PER_SUB // CHUNK
ROWS_PER_SUB = N // NUM_SUB
NPAD = 10112
KCHUNK = 40
KEDGES_PER_SUB = E // 2 // NUM_SUB
NKCHUNK = KEDGES_PER_SUB // KCHUNK

BN = 1000
GRID_N = N // BN

_mesh = plsc.VectorSubcoreMesh(core_axis_name="c", subcore_axis_name="s")


def _sc_logits(es16, ed16, src, dst):
    @functools.partial(
        pl.kernel,
        out_type=[
            jax.ShapeDtypeStruct((E * 16,), F32),
            jax.ShapeDtypeStruct((N, 128), F32),
            jax.ShapeDtypeStruct((N, 128), F32),
        ],
        mesh=_mesh,
        scratch_types=[
            pltpu.VMEM((1, KCHUNK), jnp.int32),
            pltpu.VMEM((1, KCHUNK), jnp.int32),
            pltpu.VMEM((KCHUNK, 128), F32),
            pltpu.VMEM((KCHUNK, 128), F32),
            pltpu.VMEM((KCHUNK, 128), F32),
            pltpu.VMEM((KCHUNK * 16,), F32),
            pltpu.VMEM_SHARED((NPAD, 128), F32),
        ],
    )
    def k(es_h, ed_h, src_h, dst_h, exq_o, d0_o, d1_o,
          sidx, didx, esr, edr, srow, exw, acc):
        c = lax.axis_index("c")
        s = lax.axis_index("s")
        zero16 = jnp.zeros((16,), F32)
        rst = s * ROWS_PER_SUB // 8 * 8

        @pl.loop(0, KCHUNK)
        def _(i):
            for r in range(8):
                srow[i, pl.ds(r * 16, 16)] = zero16

        for k2 in range(16):
            pltpu.sync_copy(srow.at[pl.ds(0, 40)],
                            acc.at[pl.ds(rst + k2 * 40, 40)])
        plsc.subcore_barrier()

        @pl.loop(0, NKCHUNK)
        def _(kk):
            off = c * (E // 2) + s * KEDGES_PER_SUB + kk * KCHUNK
            pltpu.sync_copy(src_h.at[pl.ds(off, KCHUNK)], sidx.at[0])
            pltpu.sync_copy(dst_h.at[pl.ds(off, KCHUNK)], didx.at[0])
            pltpu.sync_copy(es_h.at[sidx.at[0]], esr)
            pltpu.sync_copy(ed_h.at[didx.at[0]], edr)

            @pl.loop(0, KCHUNK)
            def _(i):
                e = esr[i, pl.ds(0, 16)] + edr[i, pl.ds(0, 16)]
                e = jnp.maximum(e, 0.2 * e)
                ex = jnp.exp(e)
                srow[i, pl.ds(0, 16)] = ex
                exw[pl.ds(i * 16, 16)] = ex

            pltpu.sync_copy(exw, exq_o.at[pl.ds(off * 16, KCHUNK * 16)])
            pltpu.sync_copy(srow, acc.at[didx.at[0]], add=True)

        plsc.subcore_barrier()

        @pl.when(c == 0)
        def _():
            pltpu.sync_copy(acc.at[pl.ds(rst, 632)], d0_o.at[pl.ds(rst, 632)])

        @pl.when(c == 1)
        def _():
            pltpu.sync_copy(acc.at[pl.ds(rst, 632)], d1_o.at[pl.ds(rst, 632)])

    return k(es16, ed16, src, dst)


def _sc_scatter(ha, hb, exq, src, dst):
    @functools.partial(
        pl.kernel,
        out_type=[
            jax.ShapeDtypeStruct((N, 128), F32),
            jax.ShapeDtypeStruct((N, 128), F32),
        ],
        mesh=_mesh,
        scratch_types=[
            pltpu.VMEM((1, CHUNK), jnp.int32),
            pltpu.VMEM((1, CHUNK), jnp.int32),
            pltpu.VMEM((CHUNK * 16,), F32),
            pltpu.VMEM((CHUNK, 128), F32),
            pltpu.VMEM((CHUNK, 128), F32),
            pltpu.VMEM_SHARED((NPAD, 128), F32),
        ],
    )
    def k(ha_h, hb_h, exq_h, src_h, dst_h, sa_o, sb_o,
          sidx, didx, exr, hrow, srow, acc):
        c = lax.axis_index("c")
        s = lax.axis_index("s")
        zero16 = jnp.zeros((16,), F32)
        rst = s * ROWS_PER_SUB // 8 * 8

        @pl.loop(0, CHUNK)
        def _(i):
            for r in range(8):
                hrow[i, pl.ds(r * 16, 16)] = zero16

        for k2 in range(8):
            pltpu.sync_copy(hrow, acc.at[pl.ds(rst + k2 * CHUNK, CHUNK)])
        plsc.subcore_barrier()

        idx0 = jnp.full((16,), 2 * c, jnp.int32)
        idx1 = idx0 + 1

        @pl.loop(0, NCHUNK)
        def _(kk):
            off = s * EDGES_PER_SUB + kk * CHUNK
            pltpu.sync_copy(src_h.at[pl.ds(off, CHUNK)], sidx.at[0])
            pltpu.sync_copy(dst_h.at[pl.ds(off, CHUNK)], didx.at[0])
            pltpu.sync_copy(exq_h.at[pl.ds(off * 16, CHUNK * 16)], exr)

            @pl.when(c == 0)
            def _():
                pltpu.sync_copy(ha_h.at[sidx.at[0]], hrow)

            @pl.when(c == 1)
            def _():
                pltpu.sync_copy(hb_h.at[sidx.at[0]], hrow)

            @pl.loop(0, CHUNK)
            def _(i):
                ex = exr[pl.ds(i * 16, 16)]
                a0 = ex.at[idx0].get(mode="promise_in_bounds")
                a1 = ex.at[idx1].get(mode="promise_in_bounds")
                for r in range(8):
                    av = a0 if r < 4 else a1
                    srow[i, pl.ds(r * 16, 16)] = hrow[i, pl.ds(r * 16, 16)] * av

            pltpu.sync_copy(srow, acc.at[didx.at[0]], add=True)

        plsc.subcore_barrier()

        @pl.when(c == 0)
        def _():
            pltpu.sync_copy(acc.at[pl.ds(rst, 632)], sa_o.at[pl.ds(rst, 632)])

        @pl.when(c == 1)
        def _():
            pltpu.sync_copy(acc.at[pl.ds(rst, 632)], sb_o.at[pl.ds(rst, 632)])

    return k(ha, hb, exq, src, dst)


def _tc_dense(x, wcat, as16, ad16):
    def body(x_ref, w_ref, as_ref, ad_ref, ha_ref, hb_ref, es_ref, ed_ref):
        h = jnp.dot(x_ref[...], w_ref[...], preferred_element_type=F32)
        ha_ref[...] = h[:, :128]
        hb_ref[...] = h[:, 128:]
        es_ref[...] = jnp.dot(h, as_ref[...], preferred_element_type=F32)
        ed_ref[...] = jnp.dot(h, ad_ref[...], preferred_element_type=F32)

    return pl.pallas_call(
        body,
        grid=(GRID_N,),
        in_specs=[
            pl.BlockSpec((BN, D_IN), lambda i: (i, 0)),
            pl.BlockSpec((D_IN, 256), lambda i: (0, 0)),
            pl.BlockSpec((256, 128), lambda i: (0, 0)),
            pl.BlockSpec((256, 128), lambda i: (0, 0)),
        ],
        out_specs=[
            pl.BlockSpec((BN, 128), lambda i: (i, 0)),
            pl.BlockSpec((BN, 128), lambda i: (i, 0)),
            pl.BlockSpec((BN, 128), lambda i: (i, 0)),
            pl.BlockSpec((BN, 128), lambda i: (i, 0)),
        ],
        out_shape=[
            jax.ShapeDtypeStruct((N, 128), F32),
            jax.ShapeDtypeStruct((N, 128), F32),
            jax.ShapeDtypeStruct((N, 128), F32),
            jax.ShapeDtypeStruct((N, 128), F32),
        ],
    )(x, wcat, as16, ad16)


def _tc_finalize(sa, sb, d0, d1, b256):
    def body(sa_ref, sb_ref, d0_ref, d1_ref, b_ref, o_ref):
        d = d0_ref[...] + d1_ref[...]
        segs = []
        for j in range(4):
            srcr = sa_ref if j < 2 else sb_ref
            jj = j % 2
            seg = srcr[...][:, jj * 64:(jj + 1) * 64]
            segs.append(seg / (d[:, j:j + 1] + 1e-16))
        x = jnp.concatenate(segs, axis=1) + b_ref[...]
        o_ref[...] = jnp.where(x > 0, x, jnp.exp(jnp.minimum(x, 0.0)) - 1.0)

    return pl.pallas_call(
        body,
        grid=(GRID_N,),
        in_specs=[
            pl.BlockSpec((BN, 128), lambda i: (i, 0)),
            pl.BlockSpec((BN, 128), lambda i: (i, 0)),
            pl.BlockSpec((BN, 128), lambda i: (i, 0)),
            pl.BlockSpec((BN, 128), lambda i: (i, 0)),
            pl.BlockSpec((1, 256), lambda i: (0, 0)),
        ],
        out_specs=pl.BlockSpec((BN, 256), lambda i: (i, 0)),
        out_shape=jax.ShapeDtypeStruct((N, 256), F32),
    )(sa, sb, d0, d1, b256)


def _tc_pool(h0, h1, watt, batt2, uatt2, wf, bf):
    def body(h0_ref, h1_ref, wa_ref, ba_ref, ua_ref, wf_ref, bf_ref,
             bot_ref, fin_ref):
        h0b = h0_ref[...]
        h1b = h1_ref[...]
        wa = wa_ref[...]
        ba = ba_ref[...]
        ua = ua_ref[...]
        v0 = jnp.tanh(jnp.dot(h0b, wa, preferred_element_type=F32) + ba)
        v1 = jnp.tanh(jnp.dot(h1b, wa, preferred_element_type=F32) + ba)
        vu0 = jnp.dot(v0, ua, preferred_element_type=F32)
        vu1 = jnp.dot(v1, ua, preferred_element_type=F32)
        m = jnp.maximum(vu0, vu1)
        e0 = jnp.exp(vu0 - m)
        e1 = jnp.exp(vu1 - m)
        tot = e0 + e1
        fin = (e0 / tot) * h0b + (e1 / tot) * h1b
        fin_ref[...] = fin
        wfm = jnp.mean(wf_ref[...], axis=0)
        bfm = jnp.mean(bf_ref[...], axis=0, keepdims=True)
        bot_ref[...] = jnp.dot(fin, wfm, preferred_element_type=F32) + bfm

    return pl.pallas_call(
        body,
        grid=(GRID_N,),
        in_specs=[
            pl.BlockSpec((BN, 256), lambda i: (i, 0)),
            pl.BlockSpec((BN, 256), lambda i: (i, 0)),
            pl.BlockSpec((256, ATT), lambda i: (0, 0)),
            pl.BlockSpec((1, ATT), lambda i: (0, 0)),
            pl.BlockSpec((ATT, 1), lambda i: (0, 0)),
            pl.BlockSpec((HEADS, 256, BOTTLE), lambda i: (0, 0, 0)),
            pl.BlockSpec((HEADS, BOTTLE), lambda i: (0, 0)),
        ],
        out_specs=[
            pl.BlockSpec((BN, BOTTLE), lambda i: (i, 0)),
            pl.BlockSpec((BN, 256), lambda i: (i, 0)),
        ],
        out_shape=[
            jax.ShapeDtypeStruct((N, BOTTLE), F32),
            jax.ShapeDtypeStruct((N, 256), F32),
        ],
    )(h0, h1, watt, batt2, uatt2, wf, bf)


def _pack_att(a):
    oh = jax.nn.one_hot(jnp.arange(HEADS), 128, dtype=F32)
    return (a[:, :, None] * oh[:, None, :]).reshape(HEADS * HID, 128)


def kernel(data, edge_index_list, W1, a1s, a1d, b1, W2, a2s, a2d, b2,
           Watt, batt, uatt, Wf, bf):
    w1cat = jnp.transpose(W1, (1, 0, 2)).reshape(D_IN, HEADS * HID)
    w2cat = jnp.transpose(W2, (1, 0, 2)).reshape(HEADS * HID, HEADS * HID)
    as1, ad1 = _pack_att(a1s), _pack_att(a1d)
    as2, ad2 = _pack_att(a2s), _pack_att(a2d)
    b1r = b1.reshape(1, HEADS * HID)
    b2r = b2.reshape(1, HEADS * HID)

    ha1, hb1, es1, ed1 = _tc_dense(data, w1cat, as1, ad1)

    h2 = []
    for p in range(P):
        src = edge_index_list[p, 0]
        dst = edge_index_list[p, 1]
        exq, d0, d1 = _sc_logits(es1, ed1, src, dst)
        sa, sb = _sc_scatter(ha1, hb1, exq, src, dst)
        h1p = _tc_finalize(sa, sb, d0, d1, b1r)
        ha2, hb2, es2, ed2 = _tc_dense(h1p, w2cat, as2, ad2)
        exq2, d02, d12 = _sc_logits(es2, ed2, src, dst)
        sa2, sb2 = _sc_scatter(ha2, hb2, exq2, src, dst)
        h2.append(_tc_finalize(sa2, sb2, d02, d12, b2r))

    bottle, final = _tc_pool(h2[0], h2[1], Watt, batt.reshape(1, ATT),
                             uatt.reshape(ATT, 1), Wf, bf)
    return (bottle, final)


# speedup vs baseline: 29.5056x; 1.4716x over previous
import functools

import jax
import jax.numpy as jnp
from jax import lax
from jax.experimental import pallas as pl
from jax.experimental.pallas import tpu as pltpu
from jax.experimental.pallas import tpu_sc as plsc

N = 10000
E = 160000
D_IN = 256
HID = 64
HEADS = 4
P = 2
BOTTLE = 64
ATT = 64

F32 = jnp.float32

NUM_CORES = 2
NUM_SUB = 16
CHUNK = 80
EDGES_PER_SUB = E // NUM_SUB
NCHUNK = EDGES_PER_SUB // CHUNK
ROWS_PER_SUB = N // NUM_SUB
NPAD = 10112
KCHUNK = 40
KEDGES_PER_SUB = E // 2 // NUM_SUB
NKCHUNK = KEDGES_PER_SUB // KCHUNK

BN = 1000
GRID_N = N // BN

_mesh = plsc.VectorSubcoreMesh(core_axis_name="c", subcore_axis_name="s")


def _sc_logits(es16, ed16, src, dst):
    @functools.partial(
        pl.kernel,
        out_type=[
            jax.ShapeDtypeStruct((E * 16,), F32),
            jax.ShapeDtypeStruct((N, 128), F32),
            jax.ShapeDtypeStruct((N, 128), F32),
        ],
        mesh=_mesh,
        scratch_types=[
            pltpu.VMEM((1, KCHUNK), jnp.int32),
            pltpu.VMEM((1, KCHUNK), jnp.int32),
            pltpu.VMEM((KCHUNK, 128), F32),
            pltpu.VMEM((KCHUNK, 128), F32),
            pltpu.VMEM((KCHUNK, 128), F32),
            pltpu.VMEM((KCHUNK * 16,), F32),
            pltpu.VMEM_SHARED((NPAD, 128), F32),
        ],
    )
    def k(es_h, ed_h, src_h, dst_h, exq_o, d0_o, d1_o,
          sidx, didx, esr, edr, srow, exw, acc):
        c = lax.axis_index("c")
        s = lax.axis_index("s")
        zero16 = jnp.zeros((16,), F32)
        rst = s * ROWS_PER_SUB // 8 * 8

        @pl.loop(0, KCHUNK)
        def _(i):
            for r in range(8):
                srow[i, pl.ds(r * 16, 16)] = zero16

        for k2 in range(16):
            pltpu.sync_copy(srow.at[pl.ds(0, 40)],
                            acc.at[pl.ds(rst + k2 * 40, 40)])
        plsc.subcore_barrier()

        @pl.loop(0, NKCHUNK)
        def _(kk):
            off = c * (E // 2) + s * KEDGES_PER_SUB + kk * KCHUNK
            pltpu.sync_copy(src_h.at[pl.ds(off, KCHUNK)], sidx.at[0])
            pltpu.sync_copy(dst_h.at[pl.ds(off, KCHUNK)], didx.at[0])
            pltpu.sync_copy(es_h.at[sidx.at[0]], esr)
            pltpu.sync_copy(ed_h.at[didx.at[0]], edr)

            @pl.loop(0, KCHUNK)
            def _(i):
                e = esr[i, pl.ds(0, 16)] + edr[i, pl.ds(0, 16)]
                e = jnp.maximum(e, 0.2 * e)
                ex = jnp.exp(e)
                srow[i, pl.ds(0, 16)] = ex
                exw[pl.ds(i * 16, 16)] = ex

            pltpu.sync_copy(exw, exq_o.at[pl.ds(off * 16, KCHUNK * 16)])
            pltpu.sync_copy(srow, acc.at[didx.at[0]], add=True)

        plsc.subcore_barrier()

        @pl.when(c == 0)
        def _():
            pltpu.sync_copy(acc.at[pl.ds(rst, 632)], d0_o.at[pl.ds(rst, 632)])

        @pl.when(c == 1)
        def _():
            pltpu.sync_copy(acc.at[pl.ds(rst, 632)], d1_o.at[pl.ds(rst, 632)])

    return k(es16, ed16, src, dst)


def _sc_scatter(ha, hb, exq, src, dst):
    @functools.partial(
        pl.kernel,
        out_type=[
            jax.ShapeDtypeStruct((N, 128), F32),
            jax.ShapeDtypeStruct((N, 128), F32),
        ],
        mesh=_mesh,
        scratch_types=[
            pltpu.VMEM((1, CHUNK), jnp.int32),
            pltpu.VMEM((1, CHUNK), jnp.int32),
            pltpu.VMEM((1, CHUNK), jnp.int32),
            pltpu.VMEM((1, CHUNK), jnp.int32),
            pltpu.VMEM((1, CHUNK), jnp.int32),
            pltpu.VMEM((1, CHUNK), jnp.int32),
            pltpu.VMEM((CHUNK * 16,), F32),
            pltpu.VMEM((CHUNK * 16,), F32),
            pltpu.VMEM((CHUNK, 128), F32),
            pltpu.VMEM((CHUNK, 128), F32),
            pltpu.VMEM((CHUNK, 128), F32),
            pltpu.VMEM((CHUNK, 128), F32),
            pltpu.VMEM_SHARED((NPAD, 128), F32),
            pltpu.SemaphoreType.DMA,
            pltpu.SemaphoreType.DMA,
            pltpu.SemaphoreType.DMA,
            pltpu.SemaphoreType.DMA,
            pltpu.SemaphoreType.DMA,
            pltpu.SemaphoreType.DMA,
        ],
    )
    def k(ha_h, hb_h, exq_h, src_h, dst_h, sa_o, sb_o,
          sidx0, sidx1, didx0, didx1, sdix0, sdix1, exr0, exr1,
          hrow0, hrow1, srow0, srow1, acc, si0, si1, sg0, sg1, ss0, ss1):
        c = lax.axis_index("c")
        s = lax.axis_index("s")
        zero16 = jnp.zeros((16,), F32)
        rst = s * ROWS_PER_SUB // 8 * 8
        B = [(sidx0, didx0, sdix0, exr0, hrow0, srow0, si0, sg0, ss0),
             (sidx1, didx1, sdix1, exr1, hrow1, srow1, si1, sg1, ss1)]

        @pl.loop(0, CHUNK)
        def _(i):
            for r in range(8):
                hrow0[i, pl.ds(r * 16, 16)] = zero16

        for k2 in range(8):
            pltpu.sync_copy(hrow0, acc.at[pl.ds(rst + k2 * CHUNK, CHUNK)])
        plsc.subcore_barrier()

        idxv0 = jnp.full((16,), 2 * c, jnp.int32)
        idxv1 = idxv0 + 1

        def off(kk):
            return s * EDGES_PER_SUB + kk * CHUNK

        def issue_idx(kk, b):
            sidx, didx = B[b][0], B[b][1]
            sem = B[b][6]
            pltpu.async_copy(src_h.at[pl.ds(off(kk), CHUNK)], sidx.at[0], sem)
            pltpu.async_copy(dst_h.at[pl.ds(off(kk), CHUNK)], didx.at[0], sem)

        def wait_idx(b):
            sidx, didx = B[b][0], B[b][1]
            sem = B[b][6]
            pltpu.make_async_copy(src_h.at[pl.ds(0, CHUNK)], sidx.at[0], sem).wait()
            pltpu.make_async_copy(dst_h.at[pl.ds(0, CHUNK)], didx.at[0], sem).wait()

        def issue_gath(kk, b):
            sidx, exr, hrow = B[b][0], B[b][3], B[b][4]
            sem = B[b][7]
            pltpu.async_copy(exq_h.at[pl.ds(off(kk) * 16, CHUNK * 16)], exr, sem)

            @pl.when(c == 0)
            def _():
                pltpu.async_copy(ha_h.at[sidx.at[0]], hrow, sem)

            @pl.when(c == 1)
            def _():
                pltpu.async_copy(hb_h.at[sidx.at[0]], hrow, sem)

        def wait_gath(b):
            sidx, exr, hrow = B[b][0], B[b][3], B[b][4]
            sem = B[b][7]
            pltpu.make_async_copy(exq_h.at[pl.ds(0, CHUNK * 16)], exr, sem).wait()

            @pl.when(c == 0)
            def _():
                pltpu.make_async_copy(ha_h.at[sidx.at[0]], hrow, sem).wait()

            @pl.when(c == 1)
            def _():
                pltpu.make_async_copy(hb_h.at[sidx.at[0]], hrow, sem).wait()

        def compute(b):
            didx, sdix, exr, hrow, srow = (B[b][1], B[b][2], B[b][3],
                                           B[b][4], B[b][5])

            @pl.loop(0, CHUNK)
            def _(i):
                ex = exr[pl.ds(i * 16, 16)]
                a0 = ex.at[idxv0].get(mode="promise_in_bounds")
                a1 = ex.at[idxv1].get(mode="promise_in_bounds")
                for r in range(8):
                    av = a0 if r < 4 else a1
                    srow[i, pl.ds(r * 16, 16)] = hrow[i, pl.ds(r * 16, 16)] * av

            for g in range(CHUNK // 16):
                sdix[0, pl.ds(g * 16, 16)] = didx[0, pl.ds(g * 16, 16)]

        def issue_scat(b):
            sdix, srow = B[b][2], B[b][5]
            pltpu.async_copy(srow, acc.at[sdix.at[0]], B[b][8], add=True)

        def wait_scat(b):
            sdix, srow = B[b][2], B[b][5]
            pltpu.make_async_copy(srow, acc.at[sdix.at[0]], B[b][8]).wait()

        issue_idx(0, 0)
        issue_idx(1, 1)
        wait_idx(0)
        issue_gath(0, 0)
        wait_idx(1)
        issue_gath(1, 1)
        wait_gath(0)
        compute(0)
        issue_idx(2, 0)
        issue_scat(0)
        wait_idx(0)
        issue_gath(2, 0)
        wait_gath(1)
        compute(1)
        issue_idx(3, 1)
        issue_scat(1)

        @pl.loop(1, (NCHUNK - 1) // 2)
        def _(m):
            for b in range(2):
                kk = 2 * m + b
                wait_scat(b)
                wait_idx(1 - b)
                issue_gath(kk + 1, 1 - b)
                wait_gath(b)
                compute(b)

                @pl.when(kk + 2 <= NCHUNK - 1)
                def _():
                    issue_idx(kk + 2, b)

                issue_scat(b)

        wait_scat(0)
        wait_gath(0)
        compute(0)
        issue_scat(0)
        wait_scat(1)
        wait_scat(0)

        plsc.subcore_barrier()

        @pl.when(c == 0)
        def _():
            pltpu.sync_copy(acc.at[pl.ds(rst, 632)], sa_o.at[pl.ds(rst, 632)])

        @pl.when(c == 1)
        def _():
            pltpu.sync_copy(acc.at[pl.ds(rst, 632)], sb_o.at[pl.ds(rst, 632)])

    return k(ha, hb, exq, src, dst)


def _tc_dense(x, wcat, as16, ad16):
    def body(x_ref, w_ref, as_ref, ad_ref, ha_ref, hb_ref, es_ref, ed_ref):
        h = jnp.dot(x_ref[...], w_ref[...], preferred_element_type=F32)
        ha_ref[...] = h[:, :128]
        hb_ref[...] = h[:, 128:]
        es_ref[...] = jnp.dot(h, as_ref[...], preferred_element_type=F32)
        ed_ref[...] = jnp.dot(h, ad_ref[...], preferred_element_type=F32)

    return pl.pallas_call(
        body,
        grid=(GRID_N,),
        in_specs=[
            pl.BlockSpec((BN, D_IN), lambda i: (i, 0)),
            pl.BlockSpec((D_IN, 256), lambda i: (0, 0)),
            pl.BlockSpec((256, 128), lambda i: (0, 0)),
            pl.BlockSpec((256, 128), lambda i: (0, 0)),
        ],
        out_specs=[
            pl.BlockSpec((BN, 128), lambda i: (i, 0)),
            pl.BlockSpec((BN, 128), lambda i: (i, 0)),
            pl.BlockSpec((BN, 128), lambda i: (i, 0)),
            pl.BlockSpec((BN, 128), lambda i: (i, 0)),
        ],
        out_shape=[
            jax.ShapeDtypeStruct((N, 128), F32),
            jax.ShapeDtypeStruct((N, 128), F32),
            jax.ShapeDtypeStruct((N, 128), F32),
            jax.ShapeDtypeStruct((N, 128), F32),
        ],
    )(x, wcat, as16, ad16)


def _tc_finalize(sa, sb, d0, d1, b256):
    def body(sa_ref, sb_ref, d0_ref, d1_ref, b_ref, o_ref):
        d = d0_ref[...] + d1_ref[...]
        segs = []
        for j in range(4):
            srcr = sa_ref if j < 2 else sb_ref
            jj = j % 2
            seg = srcr[...][:, jj * 64:(jj + 1) * 64]
            segs.append(seg / (d[:, j:j + 1] + 1e-16))
        x = jnp.concatenate(segs, axis=1) + b_ref[...]
        o_ref[...] = jnp.where(x > 0, x, jnp.exp(jnp.minimum(x, 0.0)) - 1.0)

    return pl.pallas_call(
        body,
        grid=(GRID_N,),
        in_specs=[
            pl.BlockSpec((BN, 128), lambda i: (i, 0)),
            pl.BlockSpec((BN, 128), lambda i: (i, 0)),
            pl.BlockSpec((BN, 128), lambda i: (i, 0)),
            pl.BlockSpec((BN, 128), lambda i: (i, 0)),
            pl.BlockSpec((1, 256), lambda i: (0, 0)),
        ],
        out_specs=pl.BlockSpec((BN, 256), lambda i: (i, 0)),
        out_shape=jax.ShapeDtypeStruct((N, 256), F32),
    )(sa, sb, d0, d1, b256)


def _tc_pool(h0, h1, watt, batt2, uatt2, wf, bf):
    def body(h0_ref, h1_ref, wa_ref, ba_ref, ua_ref, wf_ref, bf_ref,
             bot_ref, fin_ref):
        h0b = h0_ref[...]
        h1b = h1_ref[...]
        wa = wa_ref[...]
        ba = ba_ref[...]
        ua = ua_ref[...]
        v0 = jnp.tanh(jnp.dot(h0b, wa, preferred_element_type=F32) + ba)
        v1 = jnp.tanh(jnp.dot(h1b, wa, preferred_element_type=F32) + ba)
        vu0 = jnp.dot(v0, ua, preferred_element_type=F32)
        vu1 = jnp.dot(v1, ua, preferred_element_type=F32)
        m = jnp.maximum(vu0, vu1)
        e0 = jnp.exp(vu0 - m)
        e1 = jnp.exp(vu1 - m)
        tot = e0 + e1
        fin = (e0 / tot) * h0b + (e1 / tot) * h1b
        fin_ref[...] = fin
        wfm = jnp.mean(wf_ref[...], axis=0)
        bfm = jnp.mean(bf_ref[...], axis=0, keepdims=True)
        bot_ref[...] = jnp.dot(fin, wfm, preferred_element_type=F32) + bfm

    return pl.pallas_call(
        body,
        grid=(GRID_N,),
        in_specs=[
            pl.BlockSpec((BN, 256), lambda i: (i, 0)),
            pl.BlockSpec((BN, 256), lambda i: (i, 0)),
            pl.BlockSpec((256, ATT), lambda i: (0, 0)),
            pl.BlockSpec((1, ATT), lambda i: (0, 0)),
            pl.BlockSpec((ATT, 1), lambda i: (0, 0)),
            pl.BlockSpec((HEADS, 256, BOTTLE), lambda i: (0, 0, 0)),
            pl.BlockSpec((HEADS, BOTTLE), lambda i: (0, 0)),
        ],
        out_specs=[
            pl.BlockSpec((BN, BOTTLE), lambda i: (i, 0)),
            pl.BlockSpec((BN, 256), lambda i: (i, 0)),
        ],
        out_shape=[
            jax.ShapeDtypeStruct((N, BOTTLE), F32),
            jax.ShapeDtypeStruct((N, 256), F32),
        ],
    )(h0, h1, watt, batt2, uatt2, wf, bf)


def _pack_att(a):
    oh = jax.nn.one_hot(jnp.arange(HEADS), 128, dtype=F32)
    return (a[:, :, None] * oh[:, None, :]).reshape(HEADS * HID, 128)


def kernel(data, edge_index_list, W1, a1s, a1d, b1, W2, a2s, a2d, b2,
           Watt, batt, uatt, Wf, bf):
    w1cat = jnp.transpose(W1, (1, 0, 2)).reshape(D_IN, HEADS * HID)
    w2cat = jnp.transpose(W2, (1, 0, 2)).reshape(HEADS * HID, HEADS * HID)
    as1, ad1 = _pack_att(a1s), _pack_att(a1d)
    as2, ad2 = _pack_att(a2s), _pack_att(a2d)
    b1r = b1.reshape(1, HEADS * HID)
    b2r = b2.reshape(1, HEADS * HID)

    ha1, hb1, es1, ed1 = _tc_dense(data, w1cat, as1, ad1)

    h2 = []
    for p in range(P):
        src = edge_index_list[p, 0]
        dst = edge_index_list[p, 1]
        exq, d0, d1 = _sc_logits(es1, ed1, src, dst)
        sa, sb = _sc_scatter(ha1, hb1, exq, src, dst)
        h1p = _tc_finalize(sa, sb, d0, d1, b1r)
        ha2, hb2, es2, ed2 = _tc_dense(h1p, w2cat, as2, ad2)
        exq2, d02, d12 = _sc_logits(es2, ed2, src, dst)
        sa2, sb2 = _sc_scatter(ha2, hb2, exq2, src, dst)
        h2.append(_tc_finalize(sa2, sb2, d02, d12, b2r))

    bottle, final = _tc_pool(h2[0], h2[1], Watt, batt.reshape(1, ATT),
                             uatt.reshape(ATT, 1), Wf, bf)
    return (bottle, final)


# speedup vs baseline: 53.2379x; 1.8043x over previous
import functools

import jax
import jax.numpy as jnp
from jax import lax
from jax.experimental import pallas as pl
from jax.experimental.pallas import tpu as pltpu
from jax.experimental.pallas import tpu_sc as plsc

N = 10000
E = 160000
D_IN = 256
HID = 64
HEADS = 4
P = 2
BOTTLE = 64
ATT = 64

F32 = jnp.float32

NUM_CORES = 2
NUM_SUB = 16
CHUNK = 80
EDGES_PER_SUB = E // NUM_SUB
NCHUNK = EDGES_PER_SUB // CHUNK
ROWS_PER_SUB = N // NUM_SUB
NPAD = 10112
KCHUNK = 40
KEDGES_PER_SUB = E // 2 // NUM_SUB
NKCHUNK = KEDGES_PER_SUB // KCHUNK

BN = 1000
GRID_N = N // BN

_mesh = plsc.VectorSubcoreMesh(core_axis_name="c", subcore_axis_name="s")


def _sc_logits(es16, ed16, src, dst):
    @functools.partial(
        pl.kernel,
        out_type=[
            jax.ShapeDtypeStruct((E * 16,), F32),
            jax.ShapeDtypeStruct((N, 128), F32),
            jax.ShapeDtypeStruct((N, 128), F32),
        ],
        mesh=_mesh,
        scratch_types=[
            pltpu.VMEM((1, KCHUNK), jnp.int32),
            pltpu.VMEM((1, KCHUNK), jnp.int32),
            pltpu.VMEM((1, KCHUNK), jnp.int32),
            pltpu.VMEM((1, KCHUNK), jnp.int32),
            pltpu.VMEM((1, KCHUNK), jnp.int32),
            pltpu.VMEM((1, KCHUNK), jnp.int32),
            pltpu.VMEM((KCHUNK, 128), F32),
            pltpu.VMEM((KCHUNK, 128), F32),
            pltpu.VMEM((KCHUNK, 128), F32),
            pltpu.VMEM((KCHUNK, 128), F32),
            pltpu.VMEM((KCHUNK, 128), F32),
            pltpu.VMEM((KCHUNK, 128), F32),
            pltpu.VMEM((KCHUNK * 16,), F32),
            pltpu.VMEM((KCHUNK * 16,), F32),
            pltpu.VMEM_SHARED((NPAD, 128), F32),
            pltpu.SemaphoreType.DMA,
            pltpu.SemaphoreType.DMA,
            pltpu.SemaphoreType.DMA,
            pltpu.SemaphoreType.DMA,
            pltpu.SemaphoreType.DMA,
            pltpu.SemaphoreType.DMA,
            pltpu.SemaphoreType.DMA,
            pltpu.SemaphoreType.DMA,
        ],
    )
    def k(es_h, ed_h, src_h, dst_h, exq_o, d0_o, d1_o,
          sidx0, sidx1, didx0, didx1, sdix0, sdix1, esr0, esr1,
          edr0, edr1, srow0, srow1, exw0, exw1, acc,
          si0, si1, sg0, sg1, ss0, ss1, sw0, sw1):
        c = lax.axis_index("c")
        s = lax.axis_index("s")
        zero16 = jnp.zeros((16,), F32)
        rst = s * ROWS_PER_SUB // 8 * 8
        B = [(sidx0, didx0, sdix0, esr0, edr0, srow0, exw0, si0, sg0, ss0, sw0),
             (sidx1, didx1, sdix1, esr1, edr1, srow1, exw1, si1, sg1, ss1, sw1)]

        @pl.loop(0, KCHUNK)
        def _(i):
            for r in range(8):
                srow0[i, pl.ds(r * 16, 16)] = zero16
                srow1[i, pl.ds(r * 16, 16)] = zero16

        for k2 in range(16):
            pltpu.sync_copy(srow0, acc.at[pl.ds(rst + k2 * KCHUNK, KCHUNK)])
        plsc.subcore_barrier()

        def off(kk):
            return c * (E // 2) + s * KEDGES_PER_SUB + kk * KCHUNK

        def issue_idx(kk, b):
            sidx, didx, sem = B[b][0], B[b][1], B[b][7]
            pltpu.async_copy(src_h.at[pl.ds(off(kk), KCHUNK)], sidx.at[0], sem)
            pltpu.async_copy(dst_h.at[pl.ds(off(kk), KCHUNK)], didx.at[0], sem)

        def wait_idx(b):
            sidx, didx, sem = B[b][0], B[b][1], B[b][7]
            pltpu.make_async_copy(src_h.at[pl.ds(0, KCHUNK)], sidx.at[0], sem).wait()
            pltpu.make_async_copy(dst_h.at[pl.ds(0, KCHUNK)], didx.at[0], sem).wait()

        def issue_gath(kk, b):
            sidx, didx, esr, edr, sem = B[b][0], B[b][1], B[b][3], B[b][4], B[b][8]
            pltpu.async_copy(es_h.at[sidx.at[0]], esr, sem)
            pltpu.async_copy(ed_h.at[didx.at[0]], edr, sem)

        def wait_gath(b):
            sidx, didx, esr, edr, sem = B[b][0], B[b][1], B[b][3], B[b][4], B[b][8]
            pltpu.make_async_copy(es_h.at[sidx.at[0]], esr, sem).wait()
            pltpu.make_async_copy(ed_h.at[didx.at[0]], edr, sem).wait()

        def compute(b):
            didx, sdix, esr, edr, srow, exw = (B[b][1], B[b][2], B[b][3],
                                               B[b][4], B[b][5], B[b][6])

            @pl.loop(0, KCHUNK)
            def _(i):
                e = esr[i, pl.ds(0, 16)] + edr[i, pl.ds(0, 16)]
                e = jnp.maximum(e, 0.2 * e)
                ex = jnp.exp(e)
                srow[i, pl.ds(0, 16)] = ex
                exw[pl.ds(i * 16, 16)] = ex

            for g0 in (0, 16, KCHUNK - 16):
                sdix[0, pl.ds(g0, 16)] = didx[0, pl.ds(g0, 16)]

        def issue_write(kk, b):
            exw, sem = B[b][6], B[b][10]
            pltpu.async_copy(exw, exq_o.at[pl.ds(off(kk) * 16, KCHUNK * 16)], sem)

        def wait_write(b):
            exw, sem = B[b][6], B[b][10]
            pltpu.make_async_copy(exw, exq_o.at[pl.ds(0, KCHUNK * 16)], sem).wait()

        def issue_scat(b):
            sdix, srow = B[b][2], B[b][5]
            pltpu.async_copy(srow, acc.at[sdix.at[0]], B[b][9], add=True)

        def wait_scat(b):
            sdix, srow = B[b][2], B[b][5]
            pltpu.make_async_copy(srow, acc.at[sdix.at[0]], B[b][9]).wait()

        issue_idx(0, 0)
        issue_idx(1, 1)
        wait_idx(0)
        issue_gath(0, 0)
        wait_idx(1)
        issue_gath(1, 1)
        wait_gath(0)
        compute(0)
        issue_idx(2, 0)
        issue_write(0, 0)
        issue_scat(0)
        wait_idx(0)
        issue_gath(2, 0)
        wait_gath(1)
        compute(1)
        issue_idx(3, 1)
        issue_write(1, 1)
        issue_scat(1)

        @pl.loop(1, (NKCHUNK - 1) // 2)
        def _(m):
            for b in range(2):
                kk = 2 * m + b
                wait_scat(b)
                wait_write(b)
                wait_idx(1 - b)
                issue_gath(kk + 1, 1 - b)
                wait_gath(b)
                compute(b)

                @pl.when(kk + 2 <= NKCHUNK - 1)
                def _():
                    issue_idx(kk + 2, b)

                issue_write(kk, b)
                issue_scat(b)

        wait_scat(0)
        wait_write(0)
        wait_gath(0)
        compute(0)
        issue_write(NKCHUNK - 1, 0)
        issue_scat(0)
        wait_scat(1)
        wait_write(1)
        wait_scat(0)
        wait_write(0)

        plsc.subcore_barrier()

        @pl.when(c == 0)
        def _():
            pltpu.sync_copy(acc.at[pl.ds(rst, 632)], d0_o.at[pl.ds(rst, 632)])

        @pl.when(c == 1)
        def _():
            pltpu.sync_copy(acc.at[pl.ds(rst, 632)], d1_o.at[pl.ds(rst, 632)])

    return k(es16, ed16, src, dst)


def _sc_scatter(ha, hb, exq, src, dst):
    @functools.partial(
        pl.kernel,
        out_type=[
            jax.ShapeDtypeStruct((N, 128), F32),
            jax.ShapeDtypeStruct((N, 128), F32),
        ],
        mesh=_mesh,
        scratch_types=[
            pltpu.VMEM((1, CHUNK), jnp.int32),
            pltpu.VMEM((1, CHUNK), jnp.int32),
            pltpu.VMEM((1, CHUNK), jnp.int32),
            pltpu.VMEM((1, CHUNK), jnp.int32),
            pltpu.VMEM((1, CHUNK), jnp.int32),
            pltpu.VMEM((1, CHUNK), jnp.int32),
            pltpu.VMEM((CHUNK * 16,), F32),
            pltpu.VMEM((CHUNK * 16,), F32),
            pltpu.VMEM((CHUNK, 128), F32),
            pltpu.VMEM((CHUNK, 128), F32),
            pltpu.VMEM((CHUNK, 128), F32),
            pltpu.VMEM((CHUNK, 128), F32),
            pltpu.VMEM_SHARED((NPAD, 128), F32),
            pltpu.SemaphoreType.DMA,
            pltpu.SemaphoreType.DMA,
            pltpu.SemaphoreType.DMA,
            pltpu.SemaphoreType.DMA,
            pltpu.SemaphoreType.DMA,
            pltpu.SemaphoreType.DMA,
        ],
    )
    def k(ha_h, hb_h, exq_h, src_h, dst_h, sa_o, sb_o,
          sidx0, sidx1, didx0, didx1, sdix0, sdix1, exr0, exr1,
          hrow0, hrow1, srow0, srow1, acc, si0, si1, sg0, sg1, ss0, ss1):
        c = lax.axis_index("c")
        s = lax.axis_index("s")
        zero16 = jnp.zeros((16,), F32)
        rst = s * ROWS_PER_SUB // 8 * 8
        B = [(sidx0, didx0, sdix0, exr0, hrow0, srow0, si0, sg0, ss0),
             (sidx1, didx1, sdix1, exr1, hrow1, srow1, si1, sg1, ss1)]

        @pl.loop(0, CHUNK)
        def _(i):
            for r in range(8):
                hrow0[i, pl.ds(r * 16, 16)] = zero16

        for k2 in range(8):
            pltpu.sync_copy(hrow0, acc.at[pl.ds(rst + k2 * CHUNK, CHUNK)])
        plsc.subcore_barrier()

        idxv0 = jnp.full((16,), 2 * c, jnp.int32)
        idxv1 = idxv0 + 1

        def off(kk):
            return s * EDGES_PER_SUB + kk * CHUNK

        def issue_idx(kk, b):
            sidx, didx = B[b][0], B[b][1]
            sem = B[b][6]
            pltpu.async_copy(src_h.at[pl.ds(off(kk), CHUNK)], sidx.at[0], sem)
            pltpu.async_copy(dst_h.at[pl.ds(off(kk), CHUNK)], didx.at[0], sem)

        def wait_idx(b):
            sidx, didx = B[b][0], B[b][1]
            sem = B[b][6]
            pltpu.make_async_copy(src_h.at[pl.ds(0, CHUNK)], sidx.at[0], sem).wait()
            pltpu.make_async_copy(dst_h.at[pl.ds(0, CHUNK)], didx.at[0], sem).wait()

        def issue_gath(kk, b):
            sidx, exr, hrow = B[b][0], B[b][3], B[b][4]
            sem = B[b][7]
            pltpu.async_copy(exq_h.at[pl.ds(off(kk) * 16, CHUNK * 16)], exr, sem)

            @pl.when(c == 0)
            def _():
                pltpu.async_copy(ha_h.at[sidx.at[0]], hrow, sem)

            @pl.when(c == 1)
            def _():
                pltpu.async_copy(hb_h.at[sidx.at[0]], hrow, sem)

        def wait_gath(b):
            sidx, exr, hrow = B[b][0], B[b][3], B[b][4]
            sem = B[b][7]
            pltpu.make_async_copy(exq_h.at[pl.ds(0, CHUNK * 16)], exr, sem).wait()

            @pl.when(c == 0)
            def _():
                pltpu.make_async_copy(ha_h.at[sidx.at[0]], hrow, sem).wait()

            @pl.when(c == 1)
            def _():
                pltpu.make_async_copy(hb_h.at[sidx.at[0]], hrow, sem).wait()

        def compute(b):
            didx, sdix, exr, hrow, srow = (B[b][1], B[b][2], B[b][3],
                                           B[b][4], B[b][5])

            @pl.loop(0, CHUNK)
            def _(i):
                ex = exr[pl.ds(i * 16, 16)]
                a0 = ex.at[idxv0].get(mode="promise_in_bounds")
                a1 = ex.at[idxv1].get(mode="promise_in_bounds")
                for r in range(8):
                    av = a0 if r < 4 else a1
                    srow[i, pl.ds(r * 16, 16)] = hrow[i, pl.ds(r * 16, 16)] * av

            for g in range(CHUNK // 16):
                sdix[0, pl.ds(g * 16, 16)] = didx[0, pl.ds(g * 16, 16)]

        def issue_scat(b):
            sdix, srow = B[b][2], B[b][5]
            pltpu.async_copy(srow, acc.at[sdix.at[0]], B[b][8], add=True)

        def wait_scat(b):
            sdix, srow = B[b][2], B[b][5]
            pltpu.make_async_copy(srow, acc.at[sdix.at[0]], B[b][8]).wait()

        issue_idx(0, 0)
        issue_idx(1, 1)
        wait_idx(0)
        issue_gath(0, 0)
        wait_idx(1)
        issue_gath(1, 1)
        wait_gath(0)
        compute(0)
        issue_idx(2, 0)
        issue_scat(0)
        wait_idx(0)
        issue_gath(2, 0)
        wait_gath(1)
        compute(1)
        issue_idx(3, 1)
        issue_scat(1)

        @pl.loop(1, (NCHUNK - 1) // 2)
        def _(m):
            for b in range(2):
                kk = 2 * m + b
                wait_scat(b)
                wait_idx(1 - b)
                issue_gath(kk + 1, 1 - b)
                wait_gath(b)
                compute(b)

                @pl.when(kk + 2 <= NCHUNK - 1)
                def _():
                    issue_idx(kk + 2, b)

                issue_scat(b)

        wait_scat(0)
        wait_gath(0)
        compute(0)
        issue_scat(0)
        wait_scat(1)
        wait_scat(0)

        plsc.subcore_barrier()

        @pl.when(c == 0)
        def _():
            pltpu.sync_copy(acc.at[pl.ds(rst, 632)], sa_o.at[pl.ds(rst, 632)])

        @pl.when(c == 1)
        def _():
            pltpu.sync_copy(acc.at[pl.ds(rst, 632)], sb_o.at[pl.ds(rst, 632)])

    return k(ha, hb, exq, src, dst)


def _tc_dense(x, wcat, as16, ad16):
    def body(x_ref, w_ref, as_ref, ad_ref, ha_ref, hb_ref, es_ref, ed_ref):
        h = jnp.dot(x_ref[...], w_ref[...], preferred_element_type=F32)
        ha_ref[...] = h[:, :128]
        hb_ref[...] = h[:, 128:]
        es_ref[...] = jnp.dot(h, as_ref[...], preferred_element_type=F32)
        ed_ref[...] = jnp.dot(h, ad_ref[...], preferred_element_type=F32)

    return pl.pallas_call(
        body,
        grid=(GRID_N,),
        in_specs=[
            pl.BlockSpec((BN, D_IN), lambda i: (i, 0)),
            pl.BlockSpec((D_IN, 256), lambda i: (0, 0)),
            pl.BlockSpec((256, 128), lambda i: (0, 0)),
            pl.BlockSpec((256, 128), lambda i: (0, 0)),
        ],
        out_specs=[
            pl.BlockSpec((BN, 128), lambda i: (i, 0)),
            pl.BlockSpec((BN, 128), lambda i: (i, 0)),
            pl.BlockSpec((BN, 128), lambda i: (i, 0)),
            pl.BlockSpec((BN, 128), lambda i: (i, 0)),
        ],
        out_shape=[
            jax.ShapeDtypeStruct((N, 128), F32),
            jax.ShapeDtypeStruct((N, 128), F32),
            jax.ShapeDtypeStruct((N, 128), F32),
            jax.ShapeDtypeStruct((N, 128), F32),
        ],
    )(x, wcat, as16, ad16)


def _tc_finalize(sa, sb, d0, d1, b256):
    def body(sa_ref, sb_ref, d0_ref, d1_ref, b_ref, o_ref):
        d = d0_ref[...] + d1_ref[...]
        segs = []
        for j in range(4):
            srcr = sa_ref if j < 2 else sb_ref
            jj = j % 2
            seg = srcr[...][:, jj * 64:(jj + 1) * 64]
            segs.append(seg / (d[:, j:j + 1] + 1e-16))
        x = jnp.concatenate(segs, axis=1) + b_ref[...]
        o_ref[...] = jnp.where(x > 0, x, jnp.exp(jnp.minimum(x, 0.0)) - 1.0)

    return pl.pallas_call(
        body,
        grid=(GRID_N,),
        in_specs=[
            pl.BlockSpec((BN, 128), lambda i: (i, 0)),
            pl.BlockSpec((BN, 128), lambda i: (i, 0)),
            pl.BlockSpec((BN, 128), lambda i: (i, 0)),
            pl.BlockSpec((BN, 128), lambda i: (i, 0)),
            pl.BlockSpec((1, 256), lambda i: (0, 0)),
        ],
        out_specs=pl.BlockSpec((BN, 256), lambda i: (i, 0)),
        out_shape=jax.ShapeDtypeStruct((N, 256), F32),
    )(sa, sb, d0, d1, b256)


def _tc_pool(h0, h1, watt, batt2, uatt2, wf, bf):
    def body(h0_ref, h1_ref, wa_ref, ba_ref, ua_ref, wf_ref, bf_ref,
             bot_ref, fin_ref):
        h0b = h0_ref[...]
        h1b = h1_ref[...]
        wa = wa_ref[...]
        ba = ba_ref[...]
        ua = ua_ref[...]
        v0 = jnp.tanh(jnp.dot(h0b, wa, preferred_element_type=F32) + ba)
        v1 = jnp.tanh(jnp.dot(h1b, wa, preferred_element_type=F32) + ba)
        vu0 = jnp.dot(v0, ua, preferred_element_type=F32)
        vu1 = jnp.dot(v1, ua, preferred_element_type=F32)
        m = jnp.maximum(vu0, vu1)
        e0 = jnp.exp(vu0 - m)
        e1 = jnp.exp(vu1 - m)
        tot = e0 + e1
        fin = (e0 / tot) * h0b + (e1 / tot) * h1b
        fin_ref[...] = fin
        wfm = jnp.mean(wf_ref[...], axis=0)
        bfm = jnp.mean(bf_ref[...], axis=0, keepdims=True)
        bot_ref[...] = jnp.dot(fin, wfm, preferred_element_type=F32) + bfm

    return pl.pallas_call(
        body,
        grid=(GRID_N,),
        in_specs=[
            pl.BlockSpec((BN, 256), lambda i: (i, 0)),
            pl.BlockSpec((BN, 256), lambda i: (i, 0)),
            pl.BlockSpec((256, ATT), lambda i: (0, 0)),
            pl.BlockSpec((1, ATT), lambda i: (0, 0)),
            pl.BlockSpec((ATT, 1), lambda i: (0, 0)),
            pl.BlockSpec((HEADS, 256, BOTTLE), lambda i: (0, 0, 0)),
            pl.BlockSpec((HEADS, BOTTLE), lambda i: (0, 0)),
        ],
        out_specs=[
            pl.BlockSpec((BN, BOTTLE), lambda i: (i, 0)),
            pl.BlockSpec((BN, 256), lambda i: (i, 0)),
        ],
        out_shape=[
            jax.ShapeDtypeStruct((N, BOTTLE), F32),
            jax.ShapeDtypeStruct((N, 256), F32),
        ],
    )(h0, h1, watt, batt2, uatt2, wf, bf)


def _pack_att(a):
    oh = jax.nn.one_hot(jnp.arange(HEADS), 128, dtype=F32)
    return (a[:, :, None] * oh[:, None, :]).reshape(HEADS * HID, 128)


def kernel(data, edge_index_list, W1, a1s, a1d, b1, W2, a2s, a2d, b2,
           Watt, batt, uatt, Wf, bf):
    w1cat = jnp.transpose(W1, (1, 0, 2)).reshape(D_IN, HEADS * HID)
    w2cat = jnp.transpose(W2, (1, 0, 2)).reshape(HEADS * HID, HEADS * HID)
    as1, ad1 = _pack_att(a1s), _pack_att(a1d)
    as2, ad2 = _pack_att(a2s), _pack_att(a2d)
    b1r = b1.reshape(1, HEADS * HID)
    b2r = b2.reshape(1, HEADS * HID)

    ha1, hb1, es1, ed1 = _tc_dense(data, w1cat, as1, ad1)

    h2 = []
    for p in range(P):
        src = edge_index_list[p, 0]
        dst = edge_index_list[p, 1]
        exq, d0, d1 = _sc_logits(es1, ed1, src, dst)
        sa, sb = _sc_scatter(ha1, hb1, exq, src, dst)
        h1p = _tc_finalize(sa, sb, d0, d1, b1r)
        ha2, hb2, es2, ed2 = _tc_dense(h1p, w2cat, as2, ad2)
        exq2, d02, d12 = _sc_logits(es2, ed2, src, dst)
        sa2, sb2 = _sc_scatter(ha2, hb2, exq2, src, dst)
        h2.append(_tc_finalize(sa2, sb2, d02, d12, b2r))

    bottle, final = _tc_pool(h2[0], h2[1], Watt, batt.reshape(1, ATT),
                             uatt.reshape(ATT, 1), Wf, bf)
    return (bottle, final)


# speedup vs baseline: 62.1204x; 1.1668x over previous
import functools

import jax
import jax.numpy as jnp
from jax import lax
from jax.experimental import pallas as pl
from jax.experimental.pallas import tpu as pltpu
from jax.experimental.pallas import tpu_sc as plsc

N = 10000
E = 160000
D_IN = 256
HID = 64
HEADS = 4
P = 2
BOTTLE = 64
ATT = 64

F32 = jnp.float32

NUM_CORES = 2
NUM_SUB = 16
CHUNK = 80
EDGES_PER_SUB = E // NUM_SUB
NCHUNK = EDGES_PER_SUB // CHUNK
ROWS_PER_SUB = N // NUM_SUB
NPAD = 10112
KCHUNK = 40
KEDGES_PER_SUB = E // 2 // NUM_SUB
NKCHUNK = KEDGES_PER_SUB // KCHUNK

BN = 1000
GRID_N = N // BN

_mesh = plsc.VectorSubcoreMesh(core_axis_name="c", subcore_axis_name="s")


def _sc_logits(es16, ed16, src, dst):
    @functools.partial(
        pl.kernel,
        out_type=[
            jax.ShapeDtypeStruct((E * 16,), F32),
            jax.ShapeDtypeStruct((N, 128), F32),
            jax.ShapeDtypeStruct((N, 128), F32),
        ],
        mesh=_mesh,
        scratch_types=[
            pltpu.VMEM((1, KCHUNK), jnp.int32),
            pltpu.VMEM((1, KCHUNK), jnp.int32),
            pltpu.VMEM((1, KCHUNK), jnp.int32),
            pltpu.VMEM((1, KCHUNK), jnp.int32),
            pltpu.VMEM((1, KCHUNK), jnp.int32),
            pltpu.VMEM((1, KCHUNK), jnp.int32),
            pltpu.VMEM((KCHUNK, 128), F32),
            pltpu.VMEM((KCHUNK, 128), F32),
            pltpu.VMEM((KCHUNK, 128), F32),
            pltpu.VMEM((KCHUNK, 128), F32),
            pltpu.VMEM((KCHUNK, 128), F32),
            pltpu.VMEM((KCHUNK, 128), F32),
            pltpu.VMEM((KCHUNK * 16,), F32),
            pltpu.VMEM((KCHUNK * 16,), F32),
            pltpu.VMEM_SHARED((NPAD, 128), F32),
            pltpu.SemaphoreType.DMA,
            pltpu.SemaphoreType.DMA,
            pltpu.SemaphoreType.DMA,
            pltpu.SemaphoreType.DMA,
            pltpu.SemaphoreType.DMA,
            pltpu.SemaphoreType.DMA,
            pltpu.SemaphoreType.DMA,
            pltpu.SemaphoreType.DMA,
        ],
    )
    def k(es_h, ed_h, src_h, dst_h, exq_o, d0_o, d1_o,
          sidx0, sidx1, didx0, didx1, sdix0, sdix1, esr0, esr1,
          edr0, edr1, srow0, srow1, exw0, exw1, acc,
          si0, si1, sg0, sg1, ss0, ss1, sw0, sw1):
        c = lax.axis_index("c")
        s = lax.axis_index("s")
        zero16 = jnp.zeros((16,), F32)
        rst = s * ROWS_PER_SUB // 8 * 8
        B = [(sidx0, didx0, sdix0, esr0, edr0, srow0, exw0, si0, sg0, ss0, sw0),
             (sidx1, didx1, sdix1, esr1, edr1, srow1, exw1, si1, sg1, ss1, sw1)]

        @pl.loop(0, KCHUNK)
        def _(i):
            for r in range(8):
                srow0[i, pl.ds(r * 16, 16)] = zero16
                srow1[i, pl.ds(r * 16, 16)] = zero16

        for k2 in range(16):
            pltpu.sync_copy(srow0, acc.at[pl.ds(rst + k2 * KCHUNK, KCHUNK)])
        plsc.subcore_barrier()

        def off(kk):
            return c * (E // 2) + s * KEDGES_PER_SUB + kk * KCHUNK

        def issue_idx(kk, b):
            sidx, didx, sem = B[b][0], B[b][1], B[b][7]
            pltpu.async_copy(src_h.at[pl.ds(off(kk), KCHUNK)], sidx.at[0], sem)
            pltpu.async_copy(dst_h.at[pl.ds(off(kk), KCHUNK)], didx.at[0], sem)

        def wait_idx(b):
            sidx, didx, sem = B[b][0], B[b][1], B[b][7]
            pltpu.make_async_copy(src_h.at[pl.ds(0, KCHUNK)], sidx.at[0], sem).wait()
            pltpu.make_async_copy(dst_h.at[pl.ds(0, KCHUNK)], didx.at[0], sem).wait()

        def issue_gath(kk, b):
            sidx, didx, esr, edr, sem = B[b][0], B[b][1], B[b][3], B[b][4], B[b][8]
            pltpu.async_copy(es_h.at[sidx.at[0]], esr, sem)
            pltpu.async_copy(ed_h.at[didx.at[0]], edr, sem)

        def wait_gath(b):
            sidx, didx, esr, edr, sem = B[b][0], B[b][1], B[b][3], B[b][4], B[b][8]
            pltpu.make_async_copy(es_h.at[sidx.at[0]], esr, sem).wait()
            pltpu.make_async_copy(ed_h.at[didx.at[0]], edr, sem).wait()

        def compute(b):
            didx, sdix, esr, edr, srow, exw = (B[b][1], B[b][2], B[b][3],
                                               B[b][4], B[b][5], B[b][6])

            @plsc.parallel_loop(0, KCHUNK, unroll=4)
            def _(i):
                e = esr[i, pl.ds(0, 16)] + edr[i, pl.ds(0, 16)]
                e = jnp.maximum(e, 0.2 * e)
                ex = jnp.exp(e)
                srow[i, pl.ds(0, 16)] = ex
                exw[pl.ds(i * 16, 16)] = ex

            for g0 in (0, 16, KCHUNK - 16):
                sdix[0, pl.ds(g0, 16)] = didx[0, pl.ds(g0, 16)]

        def issue_write(kk, b):
            exw, sem = B[b][6], B[b][10]
            pltpu.async_copy(exw, exq_o.at[pl.ds(off(kk) * 16, KCHUNK * 16)], sem)

        def wait_write(b):
            exw, sem = B[b][6], B[b][10]
            pltpu.make_async_copy(exw, exq_o.at[pl.ds(0, KCHUNK * 16)], sem).wait()

        def issue_scat(b):
            sdix, srow = B[b][2], B[b][5]
            pltpu.async_copy(srow, acc.at[sdix.at[0]], B[b][9], add=True)

        def wait_scat(b):
            sdix, srow = B[b][2], B[b][5]
            pltpu.make_async_copy(srow, acc.at[sdix.at[0]], B[b][9]).wait()

        issue_idx(0, 0)
        issue_idx(1, 1)
        wait_idx(0)
        issue_gath(0, 0)
        wait_idx(1)
        issue_gath(1, 1)
        wait_gath(0)
        compute(0)
        issue_idx(2, 0)
        issue_write(0, 0)
        issue_scat(0)
        wait_idx(0)
        issue_gath(2, 0)
        wait_gath(1)
        compute(1)
        issue_idx(3, 1)
        issue_write(1, 1)
        issue_scat(1)

        @pl.loop(1, (NKCHUNK - 1) // 2)
        def _(m):
            for b in range(2):
                kk = 2 * m + b
                wait_scat(b)
                wait_write(b)
                wait_idx(1 - b)
                issue_gath(kk + 1, 1 - b)
                wait_gath(b)
                compute(b)

                @pl.when(kk + 2 <= NKCHUNK - 1)
                def _():
                    issue_idx(kk + 2, b)

                issue_write(kk, b)
                issue_scat(b)

        wait_scat(0)
        wait_write(0)
        wait_gath(0)
        compute(0)
        issue_write(NKCHUNK - 1, 0)
        issue_scat(0)
        wait_scat(1)
        wait_write(1)
        wait_scat(0)
        wait_write(0)

        plsc.subcore_barrier()

        @pl.when(c == 0)
        def _():
            pltpu.sync_copy(acc.at[pl.ds(rst, 632)], d0_o.at[pl.ds(rst, 632)])

        @pl.when(c == 1)
        def _():
            pltpu.sync_copy(acc.at[pl.ds(rst, 632)], d1_o.at[pl.ds(rst, 632)])

    return k(es16, ed16, src, dst)


def _sc_scatter(ha, hb, exq, src, dst):
    @functools.partial(
        pl.kernel,
        out_type=[
            jax.ShapeDtypeStruct((N, 128), F32),
            jax.ShapeDtypeStruct((N, 128), F32),
        ],
        mesh=_mesh,
        scratch_types=[
            pltpu.VMEM((1, CHUNK), jnp.int32),
            pltpu.VMEM((1, CHUNK), jnp.int32),
            pltpu.VMEM((1, CHUNK), jnp.int32),
            pltpu.VMEM((1, CHUNK), jnp.int32),
            pltpu.VMEM((1, CHUNK), jnp.int32),
            pltpu.VMEM((1, CHUNK), jnp.int32),
            pltpu.VMEM((CHUNK * 16,), F32),
            pltpu.VMEM((CHUNK * 16,), F32),
            pltpu.VMEM((CHUNK, 128), F32),
            pltpu.VMEM((CHUNK, 128), F32),
            pltpu.VMEM((CHUNK, 128), F32),
            pltpu.VMEM((CHUNK, 128), F32),
            pltpu.VMEM_SHARED((NPAD, 128), F32),
            pltpu.SemaphoreType.DMA,
            pltpu.SemaphoreType.DMA,
            pltpu.SemaphoreType.DMA,
            pltpu.SemaphoreType.DMA,
            pltpu.SemaphoreType.DMA,
            pltpu.SemaphoreType.DMA,
        ],
    )
    def k(ha_h, hb_h, exq_h, src_h, dst_h, sa_o, sb_o,
          sidx0, sidx1, didx0, didx1, sdix0, sdix1, exr0, exr1,
          hrow0, hrow1, srow0, srow1, acc, si0, si1, sg0, sg1, ss0, ss1):
        c = lax.axis_index("c")
        s = lax.axis_index("s")
        zero16 = jnp.zeros((16,), F32)
        rst = s * ROWS_PER_SUB // 8 * 8
        B = [(sidx0, didx0, sdix0, exr0, hrow0, srow0, si0, sg0, ss0),
             (sidx1, didx1, sdix1, exr1, hrow1, srow1, si1, sg1, ss1)]

        @pl.loop(0, CHUNK)
        def _(i):
            for r in range(8):
                hrow0[i, pl.ds(r * 16, 16)] = zero16

        for k2 in range(8):
            pltpu.sync_copy(hrow0, acc.at[pl.ds(rst + k2 * CHUNK, CHUNK)])
        plsc.subcore_barrier()

        idxv0 = jnp.full((16,), 2 * c, jnp.int32)
        idxv1 = idxv0 + 1

        def off(kk):
            return s * EDGES_PER_SUB + kk * CHUNK

        def issue_idx(kk, b):
            sidx, didx = B[b][0], B[b][1]
            sem = B[b][6]
            pltpu.async_copy(src_h.at[pl.ds(off(kk), CHUNK)], sidx.at[0], sem)
            pltpu.async_copy(dst_h.at[pl.ds(off(kk), CHUNK)], didx.at[0], sem)

        def wait_idx(b):
            sidx, didx = B[b][0], B[b][1]
            sem = B[b][6]
            pltpu.make_async_copy(src_h.at[pl.ds(0, CHUNK)], sidx.at[0], sem).wait()
            pltpu.make_async_copy(dst_h.at[pl.ds(0, CHUNK)], didx.at[0], sem).wait()

        def issue_gath(kk, b):
            sidx, exr, hrow = B[b][0], B[b][3], B[b][4]
            sem = B[b][7]
            pltpu.async_copy(exq_h.at[pl.ds(off(kk) * 16, CHUNK * 16)], exr, sem)

            @pl.when(c == 0)
            def _():
                pltpu.async_copy(ha_h.at[sidx.at[0]], hrow, sem)

            @pl.when(c == 1)
            def _():
                pltpu.async_copy(hb_h.at[sidx.at[0]], hrow, sem)

        def wait_gath(b):
            sidx, exr, hrow = B[b][0], B[b][3], B[b][4]
            sem = B[b][7]
            pltpu.make_async_copy(exq_h.at[pl.ds(0, CHUNK * 16)], exr, sem).wait()

            @pl.when(c == 0)
            def _():
                pltpu.make_async_copy(ha_h.at[sidx.at[0]], hrow, sem).wait()

            @pl.when(c == 1)
            def _():
                pltpu.make_async_copy(hb_h.at[sidx.at[0]], hrow, sem).wait()

        def compute(b):
            didx, sdix, exr, hrow, srow = (B[b][1], B[b][2], B[b][3],
                                           B[b][4], B[b][5])

            @plsc.parallel_loop(0, CHUNK, unroll=4)
            def _(i):
                ex = exr[pl.ds(i * 16, 16)]
                a0 = ex.at[idxv0].get(mode="promise_in_bounds")
                a1 = ex.at[idxv1].get(mode="promise_in_bounds")
                for r in range(8):
                    av = a0 if r < 4 else a1
                    srow[i, pl.ds(r * 16, 16)] = hrow[i, pl.ds(r * 16, 16)] * av

            for g in range(CHUNK // 16):
                sdix[0, pl.ds(g * 16, 16)] = didx[0, pl.ds(g * 16, 16)]

        def issue_scat(b):
            sdix, srow = B[b][2], B[b][5]
            pltpu.async_copy(srow, acc.at[sdix.at[0]], B[b][8], add=True)

        def wait_scat(b):
            sdix, srow = B[b][2], B[b][5]
            pltpu.make_async_copy(srow, acc.at[sdix.at[0]], B[b][8]).wait()

        issue_idx(0, 0)
        issue_idx(1, 1)
        wait_idx(0)
        issue_gath(0, 0)
        wait_idx(1)
        issue_gath(1, 1)
        wait_gath(0)
        compute(0)
        issue_idx(2, 0)
        issue_scat(0)
        wait_idx(0)
        issue_gath(2, 0)
        wait_gath(1)
        compute(1)
        issue_idx(3, 1)
        issue_scat(1)

        @pl.loop(1, (NCHUNK - 1) // 2)
        def _(m):
            for b in range(2):
                kk = 2 * m + b
                wait_scat(b)
                wait_idx(1 - b)
                issue_gath(kk + 1, 1 - b)
                wait_gath(b)
                compute(b)

                @pl.when(kk + 2 <= NCHUNK - 1)
                def _():
                    issue_idx(kk + 2, b)

                issue_scat(b)

        wait_scat(0)
        wait_gath(0)
        compute(0)
        issue_scat(0)
        wait_scat(1)
        wait_scat(0)

        plsc.subcore_barrier()

        @pl.when(c == 0)
        def _():
            pltpu.sync_copy(acc.at[pl.ds(rst, 632)], sa_o.at[pl.ds(rst, 632)])

        @pl.when(c == 1)
        def _():
            pltpu.sync_copy(acc.at[pl.ds(rst, 632)], sb_o.at[pl.ds(rst, 632)])

    return k(ha, hb, exq, src, dst)


def _tc_dense(x, wcat, as16, ad16):
    def body(x_ref, w_ref, as_ref, ad_ref, ha_ref, hb_ref, es_ref, ed_ref):
        h = jnp.dot(x_ref[...], w_ref[...], preferred_element_type=F32)
        ha_ref[...] = h[:, :128]
        hb_ref[...] = h[:, 128:]
        es_ref[...] = jnp.dot(h, as_ref[...], preferred_element_type=F32)
        ed_ref[...] = jnp.dot(h, ad_ref[...], preferred_element_type=F32)

    return pl.pallas_call(
        body,
        grid=(GRID_N,),
        in_specs=[
            pl.BlockSpec((BN, D_IN), lambda i: (i, 0)),
            pl.BlockSpec((D_IN, 256), lambda i: (0, 0)),
            pl.BlockSpec((256, 128), lambda i: (0, 0)),
            pl.BlockSpec((256, 128), lambda i: (0, 0)),
        ],
        out_specs=[
            pl.BlockSpec((BN, 128), lambda i: (i, 0)),
            pl.BlockSpec((BN, 128), lambda i: (i, 0)),
            pl.BlockSpec((BN, 128), lambda i: (i, 0)),
            pl.BlockSpec((BN, 128), lambda i: (i, 0)),
        ],
        out_shape=[
            jax.ShapeDtypeStruct((N, 128), F32),
            jax.ShapeDtypeStruct((N, 128), F32),
            jax.ShapeDtypeStruct((N, 128), F32),
            jax.ShapeDtypeStruct((N, 128), F32),
        ],
    )(x, wcat, as16, ad16)


def _tc_finalize(sa, sb, d0, d1, b256):
    def body(sa_ref, sb_ref, d0_ref, d1_ref, b_ref, o_ref):
        d = d0_ref[...] + d1_ref[...]
        segs = []
        for j in range(4):
            srcr = sa_ref if j < 2 else sb_ref
            jj = j % 2
            seg = srcr[...][:, jj * 64:(jj + 1) * 64]
            segs.append(seg / (d[:, j:j + 1] + 1e-16))
        x = jnp.concatenate(segs, axis=1) + b_ref[...]
        o_ref[...] = jnp.where(x > 0, x, jnp.exp(jnp.minimum(x, 0.0)) - 1.0)

    return pl.pallas_call(
        body,
        grid=(GRID_N,),
        in_specs=[
            pl.BlockSpec((BN, 128), lambda i: (i, 0)),
            pl.BlockSpec((BN, 128), lambda i: (i, 0)),
            pl.BlockSpec((BN, 128), lambda i: (i, 0)),
            pl.BlockSpec((BN, 128), lambda i: (i, 0)),
            pl.BlockSpec((1, 256), lambda i: (0, 0)),
        ],
        out_specs=pl.BlockSpec((BN, 256), lambda i: (i, 0)),
        out_shape=jax.ShapeDtypeStruct((N, 256), F32),
    )(sa, sb, d0, d1, b256)


def _tc_pool(h0, h1, watt, batt2, uatt2, wf, bf):
    def body(h0_ref, h1_ref, wa_ref, ba_ref, ua_ref, wf_ref, bf_ref,
             bot_ref, fin_ref):
        h0b = h0_ref[...]
        h1b = h1_ref[...]
        wa = wa_ref[...]
        ba = ba_ref[...]
        ua = ua_ref[...]
        v0 = jnp.tanh(jnp.dot(h0b, wa, preferred_element_type=F32) + ba)
        v1 = jnp.tanh(jnp.dot(h1b, wa, preferred_element_type=F32) + ba)
        vu0 = jnp.dot(v0, ua, preferred_element_type=F32)
        vu1 = jnp.dot(v1, ua, preferred_element_type=F32)
        m = jnp.maximum(vu0, vu1)
        e0 = jnp.exp(vu0 - m)
        e1 = jnp.exp(vu1 - m)
        tot = e0 + e1
        fin = (e0 / tot) * h0b + (e1 / tot) * h1b
        fin_ref[...] = fin
        wfm = jnp.mean(wf_ref[...], axis=0)
        bfm = jnp.mean(bf_ref[...], axis=0, keepdims=True)
        bot_ref[...] = jnp.dot(fin, wfm, preferred_element_type=F32) + bfm

    return pl.pallas_call(
        body,
        grid=(GRID_N,),
        in_specs=[
            pl.BlockSpec((BN, 256), lambda i: (i, 0)),
            pl.BlockSpec((BN, 256), lambda i: (i, 0)),
            pl.BlockSpec((256, ATT), lambda i: (0, 0)),
            pl.BlockSpec((1, ATT), lambda i: (0, 0)),
            pl.BlockSpec((ATT, 1), lambda i: (0, 0)),
            pl.BlockSpec((HEADS, 256, BOTTLE), lambda i: (0, 0, 0)),
            pl.BlockSpec((HEADS, BOTTLE), lambda i: (0, 0)),
        ],
        out_specs=[
            pl.BlockSpec((BN, BOTTLE), lambda i: (i, 0)),
            pl.BlockSpec((BN, 256), lambda i: (i, 0)),
        ],
        out_shape=[
            jax.ShapeDtypeStruct((N, BOTTLE), F32),
            jax.ShapeDtypeStruct((N, 256), F32),
        ],
    )(h0, h1, watt, batt2, uatt2, wf, bf)


def _pack_att(a):
    oh = jax.nn.one_hot(jnp.arange(HEADS), 128, dtype=F32)
    return (a[:, :, None] * oh[:, None, :]).reshape(HEADS * HID, 128)


def kernel(data, edge_index_list, W1, a1s, a1d, b1, W2, a2s, a2d, b2,
           Watt, batt, uatt, Wf, bf):
    w1cat = jnp.transpose(W1, (1, 0, 2)).reshape(D_IN, HEADS * HID)
    w2cat = jnp.transpose(W2, (1, 0, 2)).reshape(HEADS * HID, HEADS * HID)
    as1, ad1 = _pack_att(a1s), _pack_att(a1d)
    as2, ad2 = _pack_att(a2s), _pack_att(a2d)
    b1r = b1.reshape(1, HEADS * HID)
    b2r = b2.reshape(1, HEADS * HID)

    ha1, hb1, es1, ed1 = _tc_dense(data, w1cat, as1, ad1)

    h2 = []
    for p in range(P):
        src = edge_index_list[p, 0]
        dst = edge_index_list[p, 1]
        exq, d0, d1 = _sc_logits(es1, ed1, src, dst)
        sa, sb = _sc_scatter(ha1, hb1, exq, src, dst)
        h1p = _tc_finalize(sa, sb, d0, d1, b1r)
        ha2, hb2, es2, ed2 = _tc_dense(h1p, w2cat, as2, ad2)
        exq2, d02, d12 = _sc_logits(es2, ed2, src, dst)
        sa2, sb2 = _sc_scatter(ha2, hb2, exq2, src, dst)
        h2.append(_tc_finalize(sa2, sb2, d02, d12, b2r))

    bottle, final = _tc_pool(h2[0], h2[1], Watt, batt.reshape(1, ATT),
                             uatt.reshape(ATT, 1), Wf, bf)
    return (bottle, final)
